# Initial kernel scaffold; baseline (speedup 1.0000x reference)
import jax
import jax.numpy as jnp
from jax.experimental import pallas as pl


def kernel(x, edge_index, enc_W, enc_b, W1, b1, W2, b2, W3, b3, W4, b4, dec_W, dec_b):
    raise NotImplementedError("write your pallas kernel here")


# speedup vs baseline: 13.6453x; 13.6453x over previous
import functools

import jax
import jax.numpy as jnp
from jax import lax
from jax.experimental import pallas as pl
from jax.experimental.pallas import tpu as pltpu
from jax.experimental.pallas import tpu_sc as plsc

N = 10000
E = 320000
D_IN = 128
D = 64
D_OUT = 4

NC = 2
NS = 16
NW = NC * NS
K = 128
B = -(-E // (NW * K))
EPT = B * K
EP = NW * EPT
NROWS = 10240
DUMMY = N
ZROWS = NROWS // NS
DEGW = 16


def _sc_edge_body(g_hbm, src_hbm, dst_hbm, zeros_hbm, out_hbm,
                  acc, src_v, dst_v, rows_v, zbuf_v):
    c = lax.axis_index("c")
    s = lax.axis_index("s")
    wid = c * NS + s
    pltpu.sync_copy(zeros_hbm, zbuf_v)
    pltpu.sync_copy(src_hbm.at[wid], src_v)
    pltpu.sync_copy(dst_hbm.at[wid], dst_v)
    for k in range(ZROWS // K):
        pltpu.sync_copy(zbuf_v, acc.at[pl.ds(s * ZROWS + k * K, K)])
    plsc.subcore_barrier()

    @pl.loop(0, B)
    def _(j):
        pltpu.sync_copy(g_hbm.at[src_v.at[j]], rows_v)
        pltpu.sync_copy(rows_v, acc.at[dst_v.at[j]], add=True)

    plsc.subcore_barrier()
    pltpu.sync_copy(
        acc.at[pl.ds(s * ZROWS, ZROWS)],
        out_hbm.at[c, pl.ds(s * ZROWS, ZROWS)],
    )


def _sc_degree_body(dst_hbm, zeros_hbm, ones_hbm, out_hbm,
                    acc, dst_v, ones_v, zbuf_v):
    c = lax.axis_index("c")
    s = lax.axis_index("s")
    wid = c * NS + s
    pltpu.sync_copy(zeros_hbm, zbuf_v)
    pltpu.sync_copy(ones_hbm, ones_v)
    pltpu.sync_copy(dst_hbm.at[wid], dst_v)
    for k in range(ZROWS // K):
        pltpu.sync_copy(zbuf_v, acc.at[pl.ds(s * ZROWS + k * K, K)])
    plsc.subcore_barrier()

    @pl.loop(0, B)
    def _(j):
        pltpu.sync_copy(ones_v, acc.at[dst_v.at[j]], add=True)

    plsc.subcore_barrier()
    pltpu.sync_copy(
        acc.at[pl.ds(s * ZROWS, ZROWS)],
        out_hbm.at[c, pl.ds(s * ZROWS, ZROWS)],
    )


@functools.cache
def _sc_kernels():
    mesh = plsc.VectorSubcoreMesh(
        core_axis_name="c", subcore_axis_name="s", num_cores=NC, num_subcores=NS
    )
    params = pltpu.CompilerParams(use_tc_tiling_on_sc=False)
    edge = pl.kernel(
        _sc_edge_body,
        compiler_params=params,
        out_type=jax.ShapeDtypeStruct((NC, NROWS, D), jnp.float32),
        mesh=mesh,
        scratch_types=[
            pltpu.VMEM_SHARED((NROWS, D), jnp.float32),
            pltpu.VMEM((B, K), jnp.int32),
            pltpu.VMEM((B, K), jnp.int32),
            pltpu.VMEM((K, D), jnp.float32),
            pltpu.VMEM((K, D), jnp.float32),
        ],
    )
    degree = pl.kernel(
        _sc_degree_body,
        compiler_params=params,
        out_type=jax.ShapeDtypeStruct((NC, NROWS, DEGW), jnp.float32),
        mesh=mesh,
        scratch_types=[
            pltpu.VMEM_SHARED((NROWS, DEGW), jnp.float32),
            pltpu.VMEM((B, K), jnp.int32),
            pltpu.VMEM((K, DEGW), jnp.float32),
            pltpu.VMEM((K, DEGW), jnp.float32),
        ],
    )
    return edge, degree


R = 1000
GRID = N // R

_DOT = dict(preferred_element_type=jnp.float32, precision=lax.Precision.HIGHEST)


def _enc_body(x_ref, d0_ref, d1_ref, encW_ref, encb_ref, W1_ref,
              g1_ref, dinv_ref):
    deg = d0_ref[:, 0:1] + d1_ref[:, 0:1] + 1.0
    dinv = lax.rsqrt(deg)
    z = jnp.maximum(jnp.dot(x_ref[...], encW_ref[...], **_DOT) + encb_ref[...], 0.0)
    g1_ref[...] = jnp.dot(z, W1_ref[...], **_DOT) * dinv
    dinv_ref[...] = dinv


def _layer_body(s0_ref, s1_ref, g_ref, dinv_ref, b_ref, Wn_ref, gn_ref):
    dinv = dinv_ref[...]
    h = jax.nn.gelu(dinv * (s0_ref[...] + s1_ref[...] + g_ref[...]) + b_ref[...])
    gn_ref[...] = jnp.dot(h, Wn_ref[...], **_DOT) * dinv


def _final_body(s0_ref, s1_ref, g_ref, dinv_ref, b_ref, decW_ref, decb_ref,
                out_ref):
    dinv = dinv_ref[...]
    h = jax.nn.gelu(dinv * (s0_ref[...] + s1_ref[...] + g_ref[...]) + b_ref[...])
    out_ref[...] = jnp.dot(h, decW_ref[...], **_DOT) + decb_ref[...]


def _rows(shape):
    return pl.BlockSpec((R,) + shape[1:], lambda i: (i,) + (0,) * (len(shape) - 1))


def _whole(shape):
    return pl.BlockSpec(shape, lambda i: (0,) * len(shape))


def _tc_call(body, in_arrays, out_shape):
    in_specs = [_rows(a.shape) if a.shape[0] in (N, NROWS) else _whole(a.shape)
                for a in in_arrays]
    out_specs = jax.tree.map(lambda s: _rows(s.shape), out_shape)
    return pl.pallas_call(
        body,
        grid=(GRID,),
        in_specs=in_specs,
        out_specs=out_specs,
        out_shape=out_shape,
    )(*in_arrays)


def kernel(x, edge_index, enc_W, enc_b, W1, b1, W2, b2, W3, b3, W4, b4,
           dec_W, dec_b):
    pad = EP - E
    src = jnp.concatenate([edge_index[0], jnp.zeros((pad,), jnp.int32)])
    dst = jnp.concatenate([edge_index[1], jnp.full((pad,), DUMMY, jnp.int32)])
    srcR = src.reshape(NW, B, K)
    dstR = dst.reshape(NW, B, K)

    zeros_d = jnp.zeros((K, D), jnp.float32)
    zeros_w = jnp.zeros((K, DEGW), jnp.float32)
    ones_w = jnp.ones((K, DEGW), jnp.float32)

    edge_scatter, degree = _sc_kernels()
    deg_parts = degree(dstR, zeros_w, ones_w)

    f32 = jnp.float32
    g1, dinv = _tc_call(
        _enc_body,
        [x, deg_parts[0], deg_parts[1], enc_W, enc_b.reshape(1, D), W1],
        (jax.ShapeDtypeStruct((N, D), f32), jax.ShapeDtypeStruct((N, 1), f32)),
    )

    g = g1
    for b, Wn in ((b1, W2), (b2, W3), (b3, W4)):
        s_parts = edge_scatter(g, srcR, dstR, zeros_d)
        g = _tc_call(
            _layer_body,
            [s_parts[0], s_parts[1], g, dinv, b.reshape(1, D), Wn],
            jax.ShapeDtypeStruct((N, D), f32),
        )

    s_parts = edge_scatter(g, srcR, dstR, zeros_d)
    out = _tc_call(
        _final_body,
        [s_parts[0], s_parts[1], g, dinv, b4.reshape(1, D), dec_W,
         dec_b.reshape(1, D_OUT)],
        jax.ShapeDtypeStruct((N, D_OUT), f32),
    )
    return out
